# baseline (speedup 1.0000x reference)
import functools

import jax
import jax.numpy as jnp
from jax.experimental import pallas as pl
from jax.experimental.pallas import tpu as pltpu


def _pick(div_candidates, dim):
    for c in div_candidates:
        if dim % c == 0:
            return c
    return dim


def _mm_kernel(a_ref, b_ref, bias_ref, o_ref, *, nk, relu_in):
    @pl.when(pl.program_id(1) == 0)
    def _zero():
        o_ref[...] = jnp.zeros_like(o_ref)

    av = a_ref[...]
    if relu_in:
        av = jnp.maximum(av, 0.0)
    acc = jnp.dot(av, b_ref[...], preferred_element_type=jnp.float32)
    o_ref[...] += acc.astype(o_ref.dtype)

    @pl.when(pl.program_id(1) == nk - 1)
    def _bias():
        o_ref[...] += bias_ref[...].astype(o_ref.dtype)


def _matmul(a, b, bias, relu_in=False, out_dtype=jnp.float32):
    m, k = a.shape
    _, d = b.shape
    m_blk = _pick((1000, 8), m)
    k_blk = _pick((2048, 512, 256, 128), k)
    grid = (m // m_blk, k // k_blk)
    bias2d = bias.reshape(1, d).astype(jnp.float32)
    return pl.pallas_call(
        functools.partial(_mm_kernel, nk=grid[1], relu_in=relu_in),
        grid=grid,
        in_specs=[
            pl.BlockSpec((m_blk, k_blk), lambda i, j: (i, j)),
            pl.BlockSpec((k_blk, d), lambda i, j: (j, 0)),
            pl.BlockSpec((1, d), lambda i, j: (0, 0)),
        ],
        out_specs=pl.BlockSpec((m_blk, d), lambda i, j: (i, 0)),
        out_shape=jax.ShapeDtypeStruct((m, d), out_dtype),
        compiler_params=pltpu.CompilerParams(
            dimension_semantics=("parallel", "arbitrary")
        ),
    )(a, b, bias2d)


def _gn_kernel(x_ref, w_ref, b_ref, a_ref, o_ref):
    xv = x_ref[...]
    mean = jnp.mean(xv, axis=0, keepdims=True)
    cent = xv - a_ref[...] * mean
    var = jnp.mean(cent * cent, axis=0, keepdims=True)
    o_ref[...] = w_ref[...] * cent / jnp.sqrt(var + 1e-5) + b_ref[...]


def _graph_norm(x, weight, bias, mean_scale):
    n, d = x.shape
    w2 = weight.reshape(1, d)
    b2 = bias.reshape(1, d)
    a2 = mean_scale.reshape(1, d)
    c_blk = _pick((128,), d)
    grid = (d // c_blk,)
    return pl.pallas_call(
        _gn_kernel,
        grid=grid,
        in_specs=[
            pl.BlockSpec((n, c_blk), lambda j: (0, j)),
            pl.BlockSpec((1, c_blk), lambda j: (0, j)),
            pl.BlockSpec((1, c_blk), lambda j: (0, j)),
            pl.BlockSpec((1, c_blk), lambda j: (0, j)),
        ],
        out_specs=pl.BlockSpec((n, c_blk), lambda j: (0, j)),
        out_shape=jax.ShapeDtypeStruct((n, d), jnp.float32),
        compiler_params=pltpu.CompilerParams(
            dimension_semantics=("parallel",)
        ),
    )(x, w2, b2, a2)


def kernel(x, edge_index, edge_weight, W1, b1, W2, b2, W3, b3,
           gn1_w, gn1_b, gn1_a, gn2_w, gn2_b, gn2_a):
    n = x.shape[0]
    row = edge_index[0].astype(jnp.int32)
    col = edge_index[1].astype(jnp.int32)
    ew = edge_weight.astype(jnp.float32)

    deg = jnp.zeros((n,), jnp.float32).at[col].add(ew) + 1.0
    dis = jnp.where(deg > 0, jax.lax.rsqrt(jnp.maximum(deg, 1e-12)), 0.0)
    norm = dis[row] * ew * dis[col]

    n_pad = -(-n // 2048) * 2048 if n > 2048 else n
    idx = jnp.arange(n, dtype=jnp.int32)
    S = jnp.zeros((n, n_pad), jnp.bfloat16).at[col, row].add(
        norm.astype(jnp.bfloat16))
    S = S.at[idx, idx].add((dis * dis).astype(jnp.bfloat16))

    def padk(a):
        if n_pad == n:
            return a
        return jnp.concatenate(
            [a, jnp.zeros((n_pad - n, a.shape[1]), a.dtype)], axis=0)

    bf = jnp.bfloat16
    h = _matmul(S, padk(_matmul(x, W1, jnp.zeros_like(b1), out_dtype=bf)), b1)
    h1 = _graph_norm(h, gn1_w, gn1_b, gn1_a)
    h = _matmul(S, padk(_matmul(h1, W2, jnp.zeros_like(b2), relu_in=True,
                                out_dtype=bf)), b2)
    h2 = _graph_norm(h, gn2_w, gn2_b, gn2_a)
    h3 = _matmul(S, padk(_matmul(h2, W3, jnp.zeros_like(b3), relu_in=True,
                                 out_dtype=bf)), b3)

    return jnp.concatenate([h1, h2, h3], axis=-1)


# speedup vs baseline: 1.5320x; 1.5320x over previous
import functools

import jax
import jax.numpy as jnp
from jax.experimental import pallas as pl
from jax.experimental.pallas import tpu as pltpu


def _pick(div_candidates, dim):
    for c in div_candidates:
        if dim % c == 0:
            return c
    return dim


def _mm_kernel(a_ref, b_ref, bias_ref, o_ref, *, nk, relu_in):
    @pl.when(pl.program_id(1) == 0)
    def _zero():
        o_ref[...] = jnp.zeros_like(o_ref)

    av = a_ref[...]
    if relu_in:
        av = jnp.maximum(av, 0.0)
    acc = jnp.dot(av, b_ref[...], preferred_element_type=jnp.float32)
    o_ref[...] += acc.astype(o_ref.dtype)

    @pl.when(pl.program_id(1) == nk - 1)
    def _bias():
        o_ref[...] += bias_ref[...].astype(o_ref.dtype)


def _matmul(a, b, bias, relu_in=False, out_dtype=jnp.float32):
    m, k = a.shape
    _, d = b.shape
    m_blk = _pick((1000, 8), m)
    k_blk = _pick((2048, 512, 256, 128), k)
    grid = (m // m_blk, k // k_blk)
    bias2d = bias.reshape(1, d).astype(jnp.float32)
    return pl.pallas_call(
        functools.partial(_mm_kernel, nk=grid[1], relu_in=relu_in),
        grid=grid,
        in_specs=[
            pl.BlockSpec((m_blk, k_blk), lambda i, j: (i, j)),
            pl.BlockSpec((k_blk, d), lambda i, j: (j, 0)),
            pl.BlockSpec((1, d), lambda i, j: (0, 0)),
        ],
        out_specs=pl.BlockSpec((m_blk, d), lambda i, j: (i, 0)),
        out_shape=jax.ShapeDtypeStruct((m, d), out_dtype),
        compiler_params=pltpu.CompilerParams(
            dimension_semantics=("parallel", "arbitrary")
        ),
    )(a, b, bias2d)


def _gn_kernel(x_ref, w_ref, b_ref, a_ref, o_ref):
    xv = x_ref[...]
    mean = jnp.mean(xv, axis=0, keepdims=True)
    cent = xv - a_ref[...] * mean
    var = jnp.mean(cent * cent, axis=0, keepdims=True)
    o_ref[...] = w_ref[...] * cent / jnp.sqrt(var + 1e-5) + b_ref[...]


def _graph_norm(x, weight, bias, mean_scale):
    n, d = x.shape
    w2 = weight.reshape(1, d)
    b2 = bias.reshape(1, d)
    a2 = mean_scale.reshape(1, d)
    c_blk = _pick((128,), d)
    grid = (d // c_blk,)
    return pl.pallas_call(
        _gn_kernel,
        grid=grid,
        in_specs=[
            pl.BlockSpec((n, c_blk), lambda j: (0, j)),
            pl.BlockSpec((1, c_blk), lambda j: (0, j)),
            pl.BlockSpec((1, c_blk), lambda j: (0, j)),
            pl.BlockSpec((1, c_blk), lambda j: (0, j)),
        ],
        out_specs=pl.BlockSpec((n, c_blk), lambda j: (0, j)),
        out_shape=jax.ShapeDtypeStruct((n, d), jnp.float32),
        compiler_params=pltpu.CompilerParams(
            dimension_semantics=("parallel",)
        ),
    )(x, w2, b2, a2)


def kernel(x, edge_index, edge_weight, W1, b1, W2, b2, W3, b3,
           gn1_w, gn1_b, gn1_a, gn2_w, gn2_b, gn2_a):
    n = x.shape[0]
    row = edge_index[0].astype(jnp.int32)
    col = edge_index[1].astype(jnp.int32)
    ew = edge_weight.astype(jnp.float32)

    deg = jnp.zeros((n,), jnp.float32).at[col].add(ew) + 1.0
    dis = jnp.where(deg > 0, jax.lax.rsqrt(jnp.maximum(deg, 1e-12)), 0.0)
    norm = dis[row] * ew * dis[col]

    n_pad = -(-n // 2048) * 2048 if n > 2048 else n
    idx = jnp.arange(n, dtype=jnp.int32)
    S = jnp.zeros((n, n_pad), jnp.float32).at[col, row].add(norm)
    S = S.at[idx, idx].add(dis * dis)
    S = S.astype(jnp.bfloat16)

    def padk(a):
        if n_pad == n:
            return a
        return jnp.concatenate(
            [a, jnp.zeros((n_pad - n, a.shape[1]), a.dtype)], axis=0)

    bf = jnp.bfloat16
    h = _matmul(S, padk(_matmul(x, W1, jnp.zeros_like(b1), out_dtype=bf)), b1)
    h1 = _graph_norm(h, gn1_w, gn1_b, gn1_a)
    h = _matmul(S, padk(_matmul(h1, W2, jnp.zeros_like(b2), relu_in=True,
                                out_dtype=bf)), b2)
    h2 = _graph_norm(h, gn2_w, gn2_b, gn2_a)
    h3 = _matmul(S, padk(_matmul(h2, W3, jnp.zeros_like(b3), relu_in=True,
                                 out_dtype=bf)), b3)

    return jnp.concatenate([h1, h2, h3], axis=-1)
